# baseline (speedup 1.0000x reference)
import functools
import math

import jax
import jax.numpy as jnp
from jax import lax
from jax.experimental import pallas as pl
from jax.experimental.pallas import tpu as pltpu
from jax.experimental.pallas import tpu_sc as plsc

_INFO = plsc.get_sparse_core_info()
_NC = _INFO.num_cores
_NS = _INFO.num_subcores
_NW = _NC * _NS
_L = _INFO.num_lanes

_G = 128
_GPC = 2
_CHUNK = _G * _GPC
_NBUF = 5


@functools.partial(jax.jit, static_argnames=("n_chunks",))
def _run(idx1, table, n_chunks):
    d = table.shape[1]
    b = idx1.shape[0]
    ipw = n_chunks * _CHUNK

    @functools.partial(
        pl.kernel,
        out_type=jax.ShapeDtypeStruct((b, d), jnp.float32),
        mesh=plsc.VectorSubcoreMesh(core_axis_name="c", subcore_axis_name="s"),
        scratch_types=[
            pltpu.VMEM((ipw,), jnp.int32),
            [pltpu.VMEM((_CHUNK, d), jnp.float32) for _ in range(_NBUF)],
            [pltpu.SemaphoreType.DMA for _ in range(_NBUF)],
            [pltpu.SemaphoreType.DMA for _ in range(_NBUF)],
        ],
        compiler_params=pltpu.CompilerParams(use_tc_tiling_on_sc=False),
    )
    def emb(idx_hbm, table_hbm, out_hbm, idx_v, rows, gsems, ssems):
        wid = lax.axis_index("s") * _NC + lax.axis_index("c")
        scale = jnp.float32(math.sqrt(d))
        pltpu.sync_copy(idx_hbm.at[pl.ds(wid * ipw, ipw)], idx_v)

        def fire_gathers(cc, bi):
            pltpu.async_copy(
                table_hbm.at[idx_v.at[pl.ds(cc * _CHUNK, _CHUNK)]],
                rows[bi],
                gsems[bi],
            )

        def drain_gathers(bi):
            pltpu.make_async_copy(
                table_hbm.at[pl.ds(0, _CHUNK)], rows[bi], gsems[bi]
            ).wait()

        def drain_scatter(bi):
            pltpu.make_async_copy(
                rows[bi], out_hbm.at[pl.ds(0, _CHUNK)], ssems[bi]
            ).wait()

        fire_gathers(0, 0)
        fire_gathers(1, 1)
        fire_gathers(2, 2)

        @pl.loop(0, n_chunks, step=_NBUF)
        def _step(c):
            for bi in range(_NBUF):
                cc = c + bi
                drain_gathers(bi)

                nbi = (bi + 3) % _NBUF

                @pl.when(cc + 3 < n_chunks)
                def _prefetch():
                    @pl.when(cc >= 2)
                    def _free():
                        drain_scatter(nbi)

                    fire_gathers(cc + 3, nbi)

                @plsc.parallel_loop(0, _CHUNK, unroll=8)
                def _scale(r):
                    for q in range(d // _L):
                        sl = pl.ds(q * _L, _L)
                        rows[bi][r, sl] = rows[bi][r, sl] * scale

                pltpu.async_copy(
                    rows[bi],
                    out_hbm.at[pl.ds((wid * n_chunks + cc) * _CHUNK, _CHUNK)],
                    ssems[bi],
                )

        for bi in range(_NBUF):
            drain_scatter(bi)

    return emb(idx1, table)


def kernel(x, table):
    batch, hist = x.shape
    d = table.shape[1]
    b = batch * hist
    assert b % (_NW * _CHUNK * _NBUF) == 0 and d % _L == 0
    idx1 = x.astype(jnp.int32).reshape(-1)
    n_chunks = b // (_NW * _CHUNK)
    out = _run(idx1, table, n_chunks)
    return out.reshape(batch, hist, d)


# speedup vs baseline: 1.0005x; 1.0005x over previous
import functools
import math

import jax
import jax.numpy as jnp
from jax import lax
from jax.experimental import pallas as pl
from jax.experimental.pallas import tpu as pltpu
from jax.experimental.pallas import tpu_sc as plsc

_INFO = plsc.get_sparse_core_info()
_NC = _INFO.num_cores
_NS = _INFO.num_subcores
_NW = _NC * _NS
_L = _INFO.num_lanes

_G = 128
_GPC = 2
_CHUNK = _G * _GPC
_NBUF = 4


@functools.partial(jax.jit, static_argnames=("n_chunks",))
def _run(idx2d, table, n_chunks):
    d = table.shape[1]
    b = idx2d.shape[0] * _G
    irows_pw = n_chunks * _GPC

    @functools.partial(
        pl.kernel,
        out_type=jax.ShapeDtypeStruct((b, d), jnp.float32),
        mesh=plsc.VectorSubcoreMesh(core_axis_name="c", subcore_axis_name="s"),
        scratch_types=[
            pltpu.VMEM((irows_pw, _G), jnp.int32),
            [pltpu.VMEM((_CHUNK, d), jnp.float32) for _ in range(_NBUF)],
            [pltpu.SemaphoreType.DMA for _ in range(_NBUF)],
            [pltpu.SemaphoreType.DMA for _ in range(_NBUF)],
        ],
        compiler_params=pltpu.CompilerParams(use_tc_tiling_on_sc=False),
    )
    def emb(idx_hbm, table_hbm, out_hbm, idx_v, rows, gsems, ssems):
        wid = lax.axis_index("s") * _NC + lax.axis_index("c")
        scale = jnp.float32(math.sqrt(d))
        pltpu.sync_copy(idx_hbm.at[pl.ds(wid * irows_pw, irows_pw)], idx_v)

        def fire_gathers(cc, bi):
            for j in range(_GPC):
                pltpu.async_copy(
                    table_hbm.at[idx_v.at[cc * _GPC + j]],
                    rows[bi].at[pl.ds(j * _G, _G)],
                    gsems[bi],
                )

        def drain_gathers(bi):
            pltpu.make_async_copy(
                table_hbm.at[pl.ds(0, _CHUNK)], rows[bi], gsems[bi]
            ).wait()

        def drain_scatter(bi):
            pltpu.make_async_copy(
                rows[bi], out_hbm.at[pl.ds(0, _CHUNK)], ssems[bi]
            ).wait()

        fire_gathers(0, 0)
        fire_gathers(1, 1)

        @pl.loop(0, n_chunks, step=_NBUF)
        def _step(c):
            for bi in range(_NBUF):
                cc = c + bi
                drain_gathers(bi)

                nbi = (bi + 2) % _NBUF

                @pl.when(cc + 2 < n_chunks)
                def _prefetch():
                    @pl.when(cc >= 2)
                    def _free():
                        drain_scatter(nbi)

                    fire_gathers(cc + 2, nbi)

                @plsc.parallel_loop(0, _CHUNK, unroll=8)
                def _scale(r):
                    for q in range(d // _L):
                        sl = pl.ds(q * _L, _L)
                        rows[bi][r, sl] = rows[bi][r, sl] * scale

                pltpu.async_copy(
                    rows[bi],
                    out_hbm.at[pl.ds((wid * n_chunks + cc) * _CHUNK, _CHUNK)],
                    ssems[bi],
                )

        for bi in range(_NBUF):
            drain_scatter(bi)

    return emb(idx2d, table)


def kernel(x, table):
    batch, hist = x.shape
    d = table.shape[1]
    b = batch * hist
    assert b % (_NW * _CHUNK * _NBUF) == 0 and d % _L == 0
    idx2d = x.astype(jnp.int32).reshape(b // _G, _G)
    n_chunks = b // (_NW * _CHUNK)
    out = _run(idx2d, table, n_chunks)
    return out.reshape(batch, hist, d)
